# baseline (speedup 1.0000x reference)
import jax
import jax.numpy as jnp
from jax import lax
from jax.experimental import pallas as pl
from jax.experimental.pallas import tpu as pltpu
from jax.experimental.pallas import tpu_sc as plsc

_EPS = 1e-06
_DT = 0.1
_T0 = 0.0
_NK = 512
_NP = 10000
_NF = 3
_NB = 64

_NC = 2
_NS = 16
_NW = _NC * _NS
_R = 16
_NSUB = _NP // _R
_KMAX = -(-_NSUB // _NW)
_RW = _R * _NF


def _body(cols_hbm, w0_hbm, w1_hbm, k_hbm, out_hbm,
          cols_v, w0_v, w1_v, buf, out_buf, idx_v, sem):
    cid = lax.axis_index("c")
    sid = lax.axis_index("s")
    wid = sid * _NC + cid

    pltpu.sync_copy(cols_hbm, cols_v)
    pltpu.sync_copy(w0_hbm, w0_v)
    pltpu.sync_copy(w1_hbm, w1_v)

    lanes = lax.iota(jnp.int32, 16)

    def do_chunk(k, carry):
        sub = wid + k * _NW

        @pl.when(sub < _NSUB)
        def _():
            nn = sub * _R
            pltpu.sync_copy(k_hbm.at[pl.ds(nn, _R), :], buf)

            def do_b(b, c2):
                col = cols_v[b, :]
                w0 = w0_v[b, :]
                w1 = w1_v[b, :]
                for t in range(_NF):
                    d = (t * 16) % _NF
                    q = lax.shift_right_logical((lanes + d) * 11, 5)
                    rows = q + (t * 16 - d) // _NF
                    r = (lanes + d) - q * _NF
                    c0 = col + r
                    k0 = plsc.load_gather(buf, [rows, c0])
                    k1 = plsc.load_gather(buf, [rows, c0 + _NF])
                    out_buf[b, pl.ds(t * 16, 16)] = k0 * w0 + k1 * w1
                return c2

            lax.fori_loop(0, _NB, do_b, 0)

            def do_out(b, c3):
                pltpu.sync_copy(out_buf.at[b], out_hbm.at[b * _NSUB + sub])
                return c3

            lax.fori_loop(0, _NB, do_out, 0)

        return carry

    lax.fori_loop(0, _KMAX, do_chunk, 0)


def _sc_spline(cols16, w0_16, w1_16, k2):
    mesh = plsc.VectorSubcoreMesh(core_axis_name="c", subcore_axis_name="s",
                                  num_cores=_NC, num_subcores=_NS)
    f = pl.kernel(
        _body,
        out_type=jax.ShapeDtypeStruct((_NB * _NSUB, _RW), jnp.float32),
        mesh=mesh,
        scratch_types=[
            pltpu.VMEM((_NB, 16), jnp.int32),
            pltpu.VMEM((_NB, 16), jnp.float32),
            pltpu.VMEM((_NB, 16), jnp.float32),
            pltpu.VMEM((_R, _NK * _NF), jnp.float32),
            pltpu.VMEM((_NB, _RW), jnp.float32),
            pltpu.VMEM((_NB,), jnp.int32),
            pltpu.SemaphoreType.DMA,
        ],
        compiler_params=pltpu.CompilerParams(use_tc_tiling_on_sc=False,
                                             needs_layout_passes=False),
    )
    return f(cols16, w0_16, w1_16, k2)


def kernel(timestamps, knots):
    t_hi = _T0 + _DT * (_NK - 1)
    ts = jnp.clip(timestamps, _T0 + _EPS, t_hi - _EPS)
    nt = (ts - _T0) / _DT
    s = jnp.floor(nt).astype(jnp.int32)
    u = (nt - s.astype(jnp.float32))
    cols16 = jnp.broadcast_to((s * _NF)[:, None], (_NB, 16))
    w1_16 = jnp.broadcast_to(u[:, None], (_NB, 16))
    w0_16 = 1.0 - w1_16
    k2 = knots.reshape(_NP, _NK * _NF)
    out = _sc_spline(cols16, w0_16, w1_16, k2)
    return out.reshape(_NB, _NP, _NF)


# speedup vs baseline: 1.1366x; 1.1366x over previous
import jax
import jax.numpy as jnp
from jax import lax
from jax.experimental import pallas as pl
from jax.experimental.pallas import tpu as pltpu
from jax.experimental.pallas import tpu_sc as plsc

_EPS = 1e-06
_DT = 0.1
_T0 = 0.0
_NK = 512
_NP = 10000
_NF = 3
_NB = 64

_NC = 2
_NS = 16
_NW = _NC * _NS
_R = 16
_NSUB = _NP // _R
_XTRA = _NSUB - (_NSUB // _NW) * _NW
_KMIN = _NSUB // _NW
_RW = _R * _NF
_ROWLEN = _NK * _NF


def _body(cols_hbm, w0_hbm, w1_hbm, k_hbm, out_hbm,
          cols_v, w0_v, w1_v, buf, out_buf, idx_v, sem_in, sem_out):
    cid = lax.axis_index("c")
    sid = lax.axis_index("s")
    wid = sid * _NC + cid

    pltpu.sync_copy(cols_hbm, cols_v)
    pltpu.sync_copy(w0_hbm, w0_v)
    pltpu.sync_copy(w1_hbm, w1_v)

    lanes = lax.iota(jnp.int32, 16)
    rows_t = []
    r_t = []
    for t in range(_NF):
        d = (t * 16) % _NF
        q = lax.shift_right_logical((lanes + d) * 11, 5)
        rows_t.append(q + (t * 16 - d) // _NF)
        r_t.append((lanes + d) - q * _NF)

    k0w = wid * _KMIN + jnp.minimum(wid, _XTRA)
    kend = k0w + _KMIN + jnp.where(wid < _XTRA, 1, 0)

    def issue_in(k, slot):
        pltpu.async_copy(k_hbm.at[pl.ds(k * _R, _R), :], buf.at[slot],
                         sem_in.at[slot])

    issue_in(k0w, k0w & 1)

    def do_chunk(k, carry):
        slot = k & 1

        @pl.when(k + 1 < kend)
        def _():
            issue_in(k + 1, 1 - slot)

        pltpu.make_async_copy(k_hbm.at[pl.ds(k * _R, _R), :], buf.at[slot],
                              sem_in.at[slot]).wait()
        @pl.when(k >= k0w + 2)
        def _():
            pltpu.make_async_copy(out_buf.at[slot], out_hbm.at[idx_v.at[slot]],
                                  sem_out.at[slot]).wait()

        for g in range(_NB // 16):
            idx_v[slot, pl.ds(g * 16, 16)] = (lanes + g * 16) * _NSUB + k

        bufs = buf.at[slot]

        def do_b(b, c2):
            col = cols_v[b, :]
            w0 = w0_v[b, :]
            w1 = w1_v[b, :]
            for t in range(_NF):
                c0 = col + r_t[t]
                k0 = plsc.load_gather(bufs, [rows_t[t], c0])
                k1 = plsc.load_gather(bufs, [rows_t[t], c0 + _NF])
                out_buf[slot, b, pl.ds(t * 16, 16)] = k0 * w0 + k1 * w1
            return c2

        lax.fori_loop(0, _NB, do_b, 0)
        pltpu.async_copy(out_buf.at[slot], out_hbm.at[idx_v.at[slot]],
                         sem_out.at[slot])
        return carry

    lax.fori_loop(k0w, kend, do_chunk, 0)

    for slot in range(2):
        pltpu.make_async_copy(out_buf.at[slot], out_hbm.at[idx_v.at[slot]],
                              sem_out.at[slot]).wait()


def _sc_spline(cols16, w0_16, w1_16, k2):
    mesh = plsc.VectorSubcoreMesh(core_axis_name="c", subcore_axis_name="s",
                                  num_cores=_NC, num_subcores=_NS)
    f = pl.kernel(
        _body,
        out_type=jax.ShapeDtypeStruct((_NB * _NSUB, _RW), jnp.float32),
        mesh=mesh,
        scratch_types=[
            pltpu.VMEM((_NB, 16), jnp.int32),
            pltpu.VMEM((_NB, 16), jnp.float32),
            pltpu.VMEM((_NB, 16), jnp.float32),
            pltpu.VMEM((2, _R, _ROWLEN), jnp.float32),
            pltpu.VMEM((2, _NB, _RW), jnp.float32),
            pltpu.VMEM((2, _NB), jnp.int32),
            pltpu.SemaphoreType.DMA((2,)),
            pltpu.SemaphoreType.DMA((2,)),
        ],
        compiler_params=pltpu.CompilerParams(use_tc_tiling_on_sc=False,
                                             needs_layout_passes=False),
    )
    return f(cols16, w0_16, w1_16, k2)


def kernel(timestamps, knots):
    t_hi = _T0 + _DT * (_NK - 1)
    ts = jnp.clip(timestamps, _T0 + _EPS, t_hi - _EPS)
    nt = (ts - _T0) / _DT
    s = jnp.floor(nt).astype(jnp.int32)
    u = (nt - s.astype(jnp.float32))
    cols16 = jnp.broadcast_to((s * _NF)[:, None], (_NB, 16))
    w1_16 = jnp.broadcast_to(u[:, None], (_NB, 16))
    w0_16 = 1.0 - w1_16
    k2 = knots.reshape(_NP, _ROWLEN)
    out = _sc_spline(cols16, w0_16, w1_16, k2)
    return out.reshape(_NB, _NP, _NF)
